# baseline (speedup 1.0000x reference)
import functools

import jax
import jax.numpy as jnp
from jax import lax
from jax.experimental import pallas as pl
from jax.experimental.pallas import tpu as pltpu
from jax.experimental.pallas import tpu_sc as plsc

_CHUNK = 8
_SPLIT = 2


def kernel(r_idx, kv):
    b, p2, w2, c_kv = kv.shape
    topk = r_idx.shape[2]
    total = b * p2 * topk * _SPLIT
    blk = (w2 * c_kv) // _SPLIT

    nc, ns = 2, 16
    nw = nc * ns

    lo_n, hi_n = 192, 200
    n_lo = (nw * hi_n - total) // (hi_n - lo_n)
    lo_rows = n_lo * lo_n
    hi_chunks = hi_n // _CHUNK
    lo_chunks = lo_n // _CHUNK

    kv_flat = kv.reshape(b * p2 * _SPLIT, blk)
    g_idx = r_idx + (jnp.arange(b, dtype=r_idx.dtype) * p2)[:, None, None]
    g_idx = g_idx.reshape(total // _SPLIT, 1).astype(jnp.int32)
    g_idx = (_SPLIT * g_idx + jnp.arange(_SPLIT, dtype=jnp.int32)).reshape(total)

    mesh = plsc.VectorSubcoreMesh(
        core_axis_name="c", subcore_axis_name="s",
        num_cores=nc, num_subcores=ns,
    )

    @functools.partial(
        pl.kernel,
        out_type=jax.ShapeDtypeStruct((total, blk), kv.dtype),
        mesh=mesh,
        scratch_types=[
            pltpu.VMEM((hi_n,), jnp.int32),
            pltpu.VMEM((2, _CHUNK, blk), jnp.float32),
            pltpu.SemaphoreType.DMA((2,)),
            pltpu.SemaphoreType.DMA((2,)),
        ],
    )
    def gather_rows(kv_hbm, idx_hbm, out_hbm, idx_v, buf, gsem, ssem):
        w = lax.axis_index("s") * nc + lax.axis_index("c")
        is_lo = w < n_lo
        base = jnp.where(is_lo, lo_n * w, lo_rows + hi_n * (w - n_lo))
        base = pl.multiple_of(base, 8)
        nch = jnp.where(is_lo, lo_chunks, hi_chunks)

        @pl.when(is_lo)
        def _():
            pltpu.sync_copy(
                idx_hbm.at[pl.ds(base, lo_n)], idx_v.at[pl.ds(0, lo_n)]
            )

        @pl.when(jnp.logical_not(is_lo))
        def _():
            pltpu.sync_copy(idx_hbm.at[pl.ds(base, hi_n)], idx_v)

        def gather(t):
            return pltpu.make_async_copy(
                kv_hbm.at[idx_v.at[pl.ds(_CHUNK * t, _CHUNK)]],
                buf.at[t % 2],
                gsem.at[t % 2],
            )

        def scatter(t):
            return pltpu.make_async_copy(
                buf.at[t % 2],
                out_hbm.at[pl.ds(base + _CHUNK * t, _CHUNK)],
                ssem.at[t % 2],
            )

        for c in range(hi_chunks + 1):
            if c < hi_chunks:
                def fill(c=c):
                    if c >= 2:
                        scatter(c - 2).wait()
                    gather(c).start()

                pl.when(c < nch)(fill)
            if c >= 1:
                def drainprev(c=c):
                    gather(c - 1).wait()
                    scatter(c - 1).start()

                pl.when(c - 1 < nch)(drainprev)

        @pl.when(is_lo)
        def _():
            scatter(lo_chunks - 2).wait()
            scatter(lo_chunks - 1).wait()

        @pl.when(jnp.logical_not(is_lo))
        def _():
            scatter(hi_chunks - 2).wait()
            scatter(hi_chunks - 1).wait()

    out = gather_rows(kv_flat, g_idx)
    return out.reshape(b, p2, topk, w2, c_kv)


# speedup vs baseline: 1.0034x; 1.0034x over previous
import functools

import jax
import jax.numpy as jnp
from jax import lax
from jax.experimental import pallas as pl
from jax.experimental.pallas import tpu as pltpu
from jax.experimental.pallas import tpu_sc as plsc

_CHUNK = 8
_SPLIT = 4
_NBUF = 4


def kernel(r_idx, kv):
    b, p2, w2, c_kv = kv.shape
    topk = r_idx.shape[2]
    total = b * p2 * topk * _SPLIT
    blk = (w2 * c_kv) // _SPLIT

    nc, ns = 2, 16
    nw = nc * ns

    per_w = total // nw
    n_chunks = per_w // _CHUNK

    kv_flat = kv.reshape(b * p2 * _SPLIT, blk)
    g_idx = r_idx + (jnp.arange(b, dtype=r_idx.dtype) * p2)[:, None, None]
    g_idx = g_idx.reshape(total // _SPLIT, 1).astype(jnp.int32)
    g_idx = (_SPLIT * g_idx + jnp.arange(_SPLIT, dtype=jnp.int32)).reshape(total)

    mesh = plsc.VectorSubcoreMesh(
        core_axis_name="c", subcore_axis_name="s",
        num_cores=nc, num_subcores=ns,
    )

    @functools.partial(
        pl.kernel,
        out_type=jax.ShapeDtypeStruct((total, blk), kv.dtype),
        mesh=mesh,
        scratch_types=[
            pltpu.VMEM((per_w,), jnp.int32),
            pltpu.VMEM((_NBUF, _CHUNK, blk), jnp.float32),
            pltpu.SemaphoreType.DMA((_NBUF,)),
            pltpu.SemaphoreType.DMA((_NBUF,)),
        ],
    )
    def gather_rows(kv_hbm, idx_hbm, out_hbm, idx_v, buf, gsem, ssem):
        w = lax.axis_index("s") * nc + lax.axis_index("c")
        base = pl.multiple_of(per_w * w, 8)

        pltpu.sync_copy(idx_hbm.at[pl.ds(base, per_w)], idx_v)

        def gather(t):
            return pltpu.make_async_copy(
                kv_hbm.at[idx_v.at[pl.ds(_CHUNK * t, _CHUNK)]],
                buf.at[t % _NBUF],
                gsem.at[t % _NBUF],
            )

        def scatter(t):
            return pltpu.make_async_copy(
                buf.at[t % _NBUF],
                out_hbm.at[pl.ds(base + _CHUNK * t, _CHUNK)],
                ssem.at[t % _NBUF],
            )

        for c in range(n_chunks + 1):
            if c < n_chunks:
                if c >= _NBUF:
                    scatter(c - _NBUF).wait()
                gather(c).start()
            if c >= 1:
                gather(c - 1).wait()
                scatter(c - 1).start()

        for t in range(n_chunks - _NBUF, n_chunks):
            scatter(t).wait()

    out = gather_rows(kv_flat, g_idx)
    return out.reshape(b, p2, topk, w2, c_kv)
